# baseline (speedup 1.0000x reference)
import functools

import jax
import jax.numpy as jnp
from jax.experimental import pallas as pl
from jax.experimental.pallas import tpu as pltpu

_NSPLIT = 4


def _body(cid_ref, f_ref, p_any, c_any, po_any, co_any,
          fn_vmem, ones_vmem, bulk_sem, row_sem, *, nsplit):
    C = p_any.shape[0]
    chunk = C // nsplit
    copies = []
    for k in range(nsplit):
        sl = pl.ds(k * chunk, chunk)
        cp = pltpu.make_async_copy(p_any.at[sl], po_any.at[sl], bulk_sem)
        cp.start()
        copies.append(cp)
    cc = pltpu.make_async_copy(c_any, co_any, bulk_sem)
    cc.start()
    copies.append(cc)

    f = f_ref[...]
    nrm = jnp.sqrt(jnp.sum(f * f, axis=1, keepdims=True))
    fn_vmem[...] = (f / jnp.maximum(nrm, 1e-12))[None]
    ones_vmem[...] = jnp.ones(ones_vmem.shape, jnp.int32)

    for cp in copies:
        cp.wait()

    cid = cid_ref[0]
    rp = pltpu.make_async_copy(fn_vmem, po_any.at[pl.ds(cid, 1)], row_sem)
    rp.start()
    rc = pltpu.make_async_copy(ones_vmem, co_any.at[pl.ds(cid, 1)], row_sem)
    rc.start()
    rp.wait()
    rc.wait()


def kernel(features, prototypes, counts, class_id):
    C, P, D = prototypes.shape
    n_add = min(features.shape[0], P)
    cid = jnp.asarray(class_id, jnp.int32).reshape((1,))
    feats = features[:n_add]

    grid_spec = pltpu.PrefetchScalarGridSpec(
        num_scalar_prefetch=1,
        grid=(1,),
        in_specs=[
            pl.BlockSpec((n_add, D), lambda i, cid_ref: (0, 0)),
            pl.BlockSpec(memory_space=pl.ANY),
            pl.BlockSpec(memory_space=pl.ANY),
        ],
        out_specs=[
            pl.BlockSpec(memory_space=pl.ANY),
            pl.BlockSpec(memory_space=pl.ANY),
        ],
        scratch_shapes=[
            pltpu.VMEM((1, n_add, D), jnp.float32),
            pltpu.VMEM((1, P), jnp.int32),
            pltpu.SemaphoreType.DMA,
            pltpu.SemaphoreType.DMA,
        ],
    )
    protos_out, counts_out = pl.pallas_call(
        functools.partial(_body, nsplit=_NSPLIT),
        grid_spec=grid_spec,
        out_shape=[
            jax.ShapeDtypeStruct((C, P, D), jnp.float32),
            jax.ShapeDtypeStruct((C, P), jnp.int32),
        ],
    )(cid, feats, prototypes, counts)
    return protos_out, counts_out


# speedup vs baseline: 14.9154x; 14.9154x over previous
import functools

import jax
import jax.numpy as jnp
from jax.experimental import pallas as pl
from jax.experimental.pallas import tpu as pltpu

_BLK = 40


def _body(cid_ref, f_ref, p_ref, c_ref, po_ref, co_ref, *, blk):
    i = pl.program_id(0)
    po_ref[...] = p_ref[...]
    co_ref[...] = c_ref[...]
    cid = cid_ref[0]
    base = i * blk

    @pl.when(jnp.logical_and(cid >= base, cid < base + blk))
    def _():
        f = f_ref[...]
        nrm = jnp.sqrt(jnp.sum(f * f, axis=1, keepdims=True))
        fn = f / jnp.maximum(nrm, 1e-12)
        r = cid - base
        po_ref[pl.ds(r, 1), :, :] = fn[None]
        co_ref[pl.ds(r, 1), :] = jnp.ones((1, c_ref.shape[1]), jnp.int32)


def kernel(features, prototypes, counts, class_id):
    C, P, D = prototypes.shape
    n_add = min(features.shape[0], P)
    cid = jnp.asarray(class_id, jnp.int32).reshape((1,))
    feats = features[:n_add]

    assert C % _BLK == 0
    grid = (C // _BLK,)

    grid_spec = pltpu.PrefetchScalarGridSpec(
        num_scalar_prefetch=1,
        grid=grid,
        in_specs=[
            pl.BlockSpec((n_add, D), lambda i, cid_ref: (0, 0)),
            pl.BlockSpec((_BLK, P, D), lambda i, cid_ref: (i, 0, 0)),
            pl.BlockSpec((_BLK, P), lambda i, cid_ref: (i, 0)),
        ],
        out_specs=[
            pl.BlockSpec((_BLK, P, D), lambda i, cid_ref: (i, 0, 0)),
            pl.BlockSpec((_BLK, P), lambda i, cid_ref: (i, 0)),
        ],
    )
    protos_out, counts_out = pl.pallas_call(
        functools.partial(_body, blk=_BLK),
        grid_spec=grid_spec,
        out_shape=[
            jax.ShapeDtypeStruct((C, P, D), jnp.float32),
            jax.ShapeDtypeStruct((C, P), jnp.int32),
        ],
    )(cid, feats, prototypes, counts)
    return protos_out, counts_out
